# baseline (speedup 1.0000x reference)
import jax
import jax.numpy as jnp
from jax.experimental import pallas as pl
from jax.experimental.pallas import tpu as pltpu


def _pick_block(n, target):
    for b in range(min(n, target), 0, -1):
        if n % b == 0:
            return b
    return n


def _mm_plain_kernel(x_ref, w_ref, o_ref):
    o_ref[...] = jnp.dot(x_ref[...], w_ref[...],
                         preferred_element_type=jnp.float32)


def _mm_tanh_post_kernel(x_ref, w_ref, p_ref, o_ref):
    acc = jnp.dot(x_ref[...], w_ref[...], preferred_element_type=jnp.float32)
    o_ref[...] = jnp.dot(jnp.tanh(acc), p_ref[...],
                         preferred_element_type=jnp.float32)


def _mm_tanh_post_bias_kernel(x_ref, w_ref, p_ref, b_ref, o_ref):
    acc = jnp.dot(x_ref[...], w_ref[...], preferred_element_type=jnp.float32)
    o_ref[...] = (jnp.dot(jnp.tanh(acc), p_ref[...],
                          preferred_element_type=jnp.float32)
                  + b_ref[...])


def _stage(x, w, post=None, bias=None, *, bm_target=400, interpret=False):
    m, kdim = x.shape
    h = w.shape[1]
    bm = _pick_block(m, bm_target)
    n_out = h if post is None else post.shape[1]

    in_specs = [
        pl.BlockSpec((bm, kdim), lambda i: (i, 0)),
        pl.BlockSpec((kdim, h), lambda i: (0, 0)),
    ]
    inputs = [x, w]
    if post is None:
        body = _mm_plain_kernel
    else:
        in_specs.append(pl.BlockSpec(post.shape, lambda i: (0, 0)))
        inputs.append(post)
        if bias is None:
            body = _mm_tanh_post_kernel
        else:
            in_specs.append(pl.BlockSpec(bias.shape, lambda i: (0, 0)))
            inputs.append(bias)
            body = _mm_tanh_post_bias_kernel

    return pl.pallas_call(
        body,
        grid=(m // bm,),
        in_specs=in_specs,
        out_specs=pl.BlockSpec((bm, n_out), lambda i: (i, 0)),
        out_shape=jax.ShapeDtypeStruct((m, n_out), jnp.float32),
        compiler_params=pltpu.CompilerParams(
            dimension_semantics=("arbitrary",)),
        interpret=interpret,
    )(*inputs)


def kernel(features, A0, W1, W2, Wc, bc):
    s1 = _stage(features, W1)
    s2 = _stage(A0, s1, post=W2)
    logits = _stage(A0, s2, post=Wc,
                    bias=bc.reshape(1, -1))
    return logits
